# baseline (speedup 1.0000x reference)
import functools

import jax
import jax.numpy as jnp
from jax import lax
from jax.experimental import pallas as pl
from jax.experimental.pallas import tpu as pltpu
from jax.experimental.pallas import tpu_sc as plsc

VOCAB = 1000000
EMB = 64
NUM_CORES = 2
NUM_SUBCORES = 16
NUM_WORKERS = NUM_CORES * NUM_SUBCORES

B_TOTAL = 4096 * 200
B_PER_W = B_TOTAL // NUM_WORKERS
CHUNK = 512
N_CHUNKS = B_PER_W // CHUNK
SUB = 128
N_SUB = CHUNK // SUB

_mesh = plsc.VectorSubcoreMesh(
    core_axis_name="c", subcore_axis_name="s",
    num_cores=NUM_CORES, num_subcores=NUM_SUBCORES)


@functools.partial(
    pl.kernel,
    out_type=jax.ShapeDtypeStruct((B_TOTAL, EMB), jnp.float32),
    mesh=_mesh,
    scratch_types=[
        pltpu.VMEM((CHUNK,), jnp.int32),
        pltpu.VMEM((CHUNK, EMB), jnp.float32),
        pltpu.SemaphoreType.DMA,
    ],
    compiler_params=pltpu.CompilerParams(use_tc_tiling_on_sc=False),
)
def _gather_kernel(idx_hbm, table_hbm, out_hbm, idx_v, rows_v, sem):
    wid = lax.axis_index("s") * NUM_CORES + lax.axis_index("c")
    base = wid * B_PER_W

    @pl.loop(0, N_CHUNKS)
    def _chunk(g):
        off = base + g * CHUNK
        pltpu.sync_copy(idx_hbm.at[pl.ds(off, CHUNK)], idx_v)
        copies = [
            pltpu.async_copy(
                table_hbm.at[idx_v.at[pl.ds(j * SUB, SUB)]],
                rows_v.at[pl.ds(j * SUB, SUB)],
                sem,
            )
            for j in range(N_SUB)
        ]
        for c in copies:
            c.wait()
        pltpu.sync_copy(rows_v, out_hbm.at[pl.ds(off, CHUNK)])


def kernel(x, table):
    idx = x.reshape(-1).astype(jnp.int32)
    out = _gather_kernel(idx, table)
    return out.reshape(x.shape + (EMB,))


# speedup vs baseline: 1.0460x; 1.0460x over previous
import functools

import jax
import jax.numpy as jnp
from jax import lax
from jax.experimental import pallas as pl
from jax.experimental.pallas import tpu as pltpu
from jax.experimental.pallas import tpu_sc as plsc

VOCAB = 1000000
EMB = 64
NUM_CORES = 2
NUM_SUBCORES = 16
NUM_WORKERS = NUM_CORES * NUM_SUBCORES

B_TOTAL = 4096 * 200
B_PER_W = B_TOTAL // NUM_WORKERS
CHUNK = 512
N_CHUNKS = B_PER_W // CHUNK
SUB = 128
N_SUB = CHUNK // SUB
NBUF = 3
N_PAD = ((N_CHUNKS + NBUF - 1) // NBUF) * NBUF + NBUF

_mesh = plsc.VectorSubcoreMesh(
    core_axis_name="c", subcore_axis_name="s",
    num_cores=NUM_CORES, num_subcores=NUM_SUBCORES)


@functools.partial(
    pl.kernel,
    out_type=jax.ShapeDtypeStruct((B_TOTAL, EMB), jnp.float32),
    mesh=_mesh,
    scratch_types=[
        pltpu.VMEM((B_PER_W,), jnp.int32),
        pltpu.VMEM((NBUF, CHUNK, EMB), jnp.float32),
        [pltpu.SemaphoreType.DMA] * NBUF,
        [pltpu.SemaphoreType.DMA] * NBUF,
    ],
    compiler_params=pltpu.CompilerParams(use_tc_tiling_on_sc=False),
)
def _gather_kernel(idx_hbm, table_hbm, out_hbm, idx_all, rows_v, gsems, ssems):
    wid = lax.axis_index("s") * NUM_CORES + lax.axis_index("c")
    base = wid * B_PER_W

    def fire_gather(g, b):
        for j in range(N_SUB):
            pltpu.async_copy(
                table_hbm.at[idx_all.at[pl.ds(g * CHUNK + j * SUB, SUB)]],
                rows_v.at[b].at[pl.ds(j * SUB, SUB)],
                gsems[b],
            )

    def drain_gather(b):
        for j in range(N_SUB):
            pltpu.make_async_copy(
                table_hbm.at[pl.ds(0, SUB)],
                rows_v.at[b].at[pl.ds(j * SUB, SUB)],
                gsems[b],
            ).wait()

    def fire_store(g, b):
        pltpu.async_copy(
            rows_v.at[b], out_hbm.at[pl.ds(base + g * CHUNK, CHUNK)], ssems[b])

    def drain_store(b):
        pltpu.make_async_copy(
            rows_v.at[b], out_hbm.at[pl.ds(base, CHUNK)], ssems[b]).wait()

    pltpu.sync_copy(idx_hbm.at[pl.ds(base, B_PER_W)], idx_all)

    fire_gather(0, 0)
    fire_gather(1, 1)

    @pl.loop(0, N_PAD, step=NBUF)
    def _triple(outer):
        for u in range(NBUF):
            g = outer + u
            b = u
            bn = (u + 2) % NBUF

            @pl.when(g < N_CHUNKS)
            def _():
                drain_gather(b)
                fire_store(g, b)

            @pl.when((g >= 1) & (g < N_CHUNKS + 1))
            def _():
                drain_store(bn)

            @pl.when(g + 2 < N_CHUNKS)
            def _():
                fire_gather(g + 2, bn)


def kernel(x, table):
    idx = x.reshape(-1).astype(jnp.int32)
    out = _gather_kernel(idx, table)
    return out.reshape(x.shape + (EMB,))


# speedup vs baseline: 1.0463x; 1.0003x over previous
import functools

import jax
import jax.numpy as jnp
from jax import lax
from jax.experimental import pallas as pl
from jax.experimental.pallas import tpu as pltpu
from jax.experimental.pallas import tpu_sc as plsc

VOCAB = 1000000
EMB = 64
SEQ = 200
BATCH = 4096
NUM_CORES = 2
NUM_SUBCORES = 16
NUM_WORKERS = NUM_CORES * NUM_SUBCORES

ROWS_PER_W = BATCH // NUM_WORKERS
B_PER_W = ROWS_PER_W * SEQ
XR_PER_CHUNK = 2
CHUNK = XR_PER_CHUNK * SEQ
N_CHUNKS = ROWS_PER_W // XR_PER_CHUNK
SUB = 80
N_SUB = CHUNK // SUB
NBUF = 3
N_PAD = ((N_CHUNKS + NBUF - 1) // NBUF) * NBUF + NBUF

_mesh = plsc.VectorSubcoreMesh(
    core_axis_name="c", subcore_axis_name="s",
    num_cores=NUM_CORES, num_subcores=NUM_SUBCORES)


@functools.partial(
    pl.kernel,
    out_type=jax.ShapeDtypeStruct((BATCH, SEQ, EMB), jnp.float32),
    mesh=_mesh,
    scratch_types=[
        pltpu.VMEM((B_PER_W,), jnp.int32),
        pltpu.VMEM((NBUF, CHUNK, EMB), jnp.float32),
        [pltpu.SemaphoreType.DMA] * NBUF,
        [pltpu.SemaphoreType.DMA] * NBUF,
    ],
    compiler_params=pltpu.CompilerParams(use_tc_tiling_on_sc=False),
)
def _gather_kernel(idx_hbm, table_hbm, out_hbm, idx_all, rows_v, gsems, ssems):
    wid = lax.axis_index("s") * NUM_CORES + lax.axis_index("c")
    base = wid * B_PER_W
    xbase = wid * ROWS_PER_W

    def fire_gather(g, b):
        for j in range(N_SUB):
            pltpu.async_copy(
                table_hbm.at[idx_all.at[pl.ds(g * CHUNK + j * SUB, SUB)]],
                rows_v.at[b].at[pl.ds(j * SUB, SUB)],
                gsems[b],
            )

    def drain_gather(b):
        for j in range(N_SUB):
            pltpu.make_async_copy(
                table_hbm.at[pl.ds(0, SUB)],
                rows_v.at[b].at[pl.ds(j * SUB, SUB)],
                gsems[b],
            ).wait()

    def fire_store(g, b):
        for r in range(XR_PER_CHUNK):
            pltpu.async_copy(
                rows_v.at[b].at[pl.ds(r * SEQ, SEQ)],
                out_hbm.at[xbase + g * XR_PER_CHUNK + r],
                ssems[b],
            )

    def drain_store(b):
        for r in range(XR_PER_CHUNK):
            pltpu.make_async_copy(
                rows_v.at[b].at[pl.ds(r * SEQ, SEQ)],
                out_hbm.at[0],
                ssems[b],
            ).wait()

    pltpu.sync_copy(idx_hbm.at[pl.ds(base, B_PER_W)], idx_all)

    fire_gather(0, 0)
    fire_gather(1, 1)

    @pl.loop(0, N_PAD, step=NBUF)
    def _triple(outer):
        for u in range(NBUF):
            g = outer + u
            b = u
            bn = (u + 2) % NBUF

            @pl.when(g < N_CHUNKS)
            def _():
                drain_gather(b)
                fire_store(g, b)

            @pl.when((g >= 1) & (g < N_CHUNKS + 1))
            def _():
                drain_store(bn)

            @pl.when(g + 2 < N_CHUNKS)
            def _():
                fire_gather(g + 2, bn)


def kernel(x, table):
    idx = x.reshape(-1).astype(jnp.int32)
    return _gather_kernel(idx, table)
